# Initial kernel scaffold; baseline (speedup 1.0000x reference)
import jax
import jax.numpy as jnp
from jax.experimental import pallas as pl


def kernel(x, edge_index, batch, W_in, b_in, W_g0, b_g0, W_g1, b_g1, W_g2, b_g2, W_c1, b_c1, W_c2, b_c2):
    raise NotImplementedError("write your pallas kernel here")


# speedup vs baseline: 10.5807x; 10.5807x over previous
import functools

import jax
import jax.numpy as jnp
from jax import lax
from jax.experimental import pallas as pl
from jax.experimental.pallas import tpu as pltpu
from jax.experimental.pallas import tpu_sc as plsc

N_NODES = 10000
N_EDGES = 320000
D = 128
N_GRAPHS = 16
N_CLASSES = 2

NC = 2
NS = 16
NW = NC * NS
EPW = N_EDGES // NW
EB = 80
NBLK = EPW // EB
RPT = 640
NPAD = RPT * NS
ZR = 40
FR = 80

_sc_mesh = plsc.VectorSubcoreMesh(core_axis_name="c", subcore_axis_name="s")


@functools.partial(
    pl.kernel,
    out_type=jax.ShapeDtypeStruct((NC * NPAD,), jnp.float32),
    mesh=_sc_mesh,
    scratch_types=[
        pltpu.VMEM((EB,), jnp.int32),
        pltpu.VMEM((EB,), jnp.float32),
        pltpu.VMEM((RPT,), jnp.float32),
        pltpu.VMEM_SHARED((NPAD,), jnp.float32),
    ],
)
def _sc_degree(dst_hbm, hist_hbm, idx_v, ones_v, buf_v, hist_sh):
    c = lax.axis_index("c")
    s = lax.axis_index("s")
    wid = c * NS + s
    for i in range(EB // 16):
        ones_v[pl.ds(i * 16, 16)] = jnp.full((16,), 1.0, jnp.float32)
    for i in range(RPT // 16):
        buf_v[pl.ds(i * 16, 16)] = jnp.zeros((16,), jnp.float32)
    pltpu.sync_copy(buf_v, hist_sh.at[pl.ds(s * RPT, RPT)])
    plsc.subcore_barrier()
    base = wid * EPW

    def body(j, carry):
        pltpu.sync_copy(dst_hbm.at[pl.ds(base + j * EB, EB)], idx_v)
        pltpu.sync_copy(ones_v, hist_sh.at[idx_v], add=True)
        return carry

    lax.fori_loop(0, NBLK, body, 0)
    plsc.subcore_barrier()
    pltpu.sync_copy(hist_sh.at[pl.ds(s * RPT, RPT)], buf_v)
    pltpu.sync_copy(buf_v, hist_hbm.at[pl.ds(c * NPAD + s * RPT, RPT)])


@functools.partial(
    pl.kernel,
    out_type=jax.ShapeDtypeStruct((NC, NPAD, D), jnp.float32),
    mesh=_sc_mesh,
    scratch_types=[
        pltpu.VMEM((EB,), jnp.int32),
        pltpu.VMEM((EB,), jnp.int32),
        pltpu.VMEM((EB, D), jnp.float32),
        pltpu.VMEM((ZR, D), jnp.float32),
        pltpu.VMEM_SHARED((NPAD, D), jnp.float32),
        pltpu.SemaphoreType.DMA,
    ],
)
def _sc_scatter(g_hbm, src_hbm, dst_hbm, out_hbm,
                si_v, di_v, rows_v, zrows_v, acc_sh, sem):
    c = lax.axis_index("c")
    s = lax.axis_index("s")
    wid = c * NS + s
    for r in range(ZR):
        for i in range(D // 16):
            zrows_v[r, pl.ds(i * 16, 16)] = jnp.zeros((16,), jnp.float32)
    for k in range(RPT // ZR):
        pltpu.sync_copy(zrows_v, acc_sh.at[pl.ds(s * RPT + k * ZR, ZR)])
    plsc.subcore_barrier()
    base = wid * EPW

    def body(j, carry):
        pltpu.sync_copy(src_hbm.at[pl.ds(base + j * EB, EB)], si_v)
        pltpu.sync_copy(dst_hbm.at[pl.ds(base + j * EB, EB)], di_v)
        pltpu.async_copy(g_hbm.at[si_v], rows_v, sem).wait()
        pltpu.sync_copy(rows_v, acc_sh.at[di_v], add=True)
        return carry

    lax.fori_loop(0, NBLK, body, 0)
    plsc.subcore_barrier()
    for k in range(RPT // FR):
        pltpu.sync_copy(acc_sh.at[pl.ds(s * RPT + k * FR, FR)], rows_v)
        pltpu.sync_copy(rows_v, out_hbm.at[c, pl.ds(s * RPT + k * FR, FR)])


RB = 400
NGRID = N_NODES // RB


def _dinv_of(h0, h1):
    deg = h0 + h1 + 1.0
    return lax.rsqrt(jnp.maximum(deg, 1.0))


def _tc_in_body(x_ref, win_ref, bin_ref, wg_ref, h0_ref, h1_ref,
                h_out, g_out):
    h = jnp.maximum(jnp.dot(x_ref[...], win_ref[...],
                            preferred_element_type=jnp.float32)
                    + bin_ref[...], 0.0)
    dinv = _dinv_of(h0_ref[...], h1_ref[...])
    h_out[...] = h
    g_out[...] = jnp.dot(h, wg_ref[...],
                         preferred_element_type=jnp.float32) * dinv


def _tc_mid_body(h_ref, g_ref, p0_ref, p1_ref, h0_ref, h1_ref, b_ref,
                 wn_ref, h_out, g_out):
    dinv = _dinv_of(h0_ref[...], h1_ref[...])
    agg = (p0_ref[...] + p1_ref[...] + g_ref[...]) * dinv
    h = jnp.maximum(h_ref[...] + agg + b_ref[...], 0.0)
    h_out[...] = h
    g_out[...] = jnp.dot(h, wn_ref[...],
                         preferred_element_type=jnp.float32) * dinv


def _tc_fin_body(h_ref, g_ref, p0_ref, p1_ref, h0_ref, h1_ref, b_ref,
                 batch_ref, wc1_ref, bc1_ref, wc2_ref, bc2_ref,
                 out_ref, pooled_acc):
    pid = pl.program_id(0)
    dinv = _dinv_of(h0_ref[...], h1_ref[...])
    agg = (p0_ref[...] + p1_ref[...] + g_ref[...]) * dinv
    h = jnp.maximum(h_ref[...] + agg + b_ref[...], 0.0)
    onehot = (batch_ref[...] ==
              lax.broadcasted_iota(jnp.int32, (RB, N_GRAPHS), 1)
              ).astype(jnp.float32)
    part = lax.dot_general(onehot, h, (((0,), (0,)), ((), ())),
                           preferred_element_type=jnp.float32)

    @pl.when(pid == 0)
    def _():
        pooled_acc[...] = jnp.zeros_like(pooled_acc)

    pooled_acc[...] += part

    @pl.when(pid == NGRID - 1)
    def _():
        pooled = pooled_acc[...]
        z = jnp.maximum(jnp.dot(pooled, wc1_ref[...],
                                preferred_element_type=jnp.float32)
                        + bc1_ref[...], 0.0)
        r = jnp.dot(z, wc2_ref[...],
                    preferred_element_type=jnp.float32) + bc2_ref[...]
        m = jnp.max(r, axis=-1, keepdims=True)
        e = jnp.exp(r - m)
        out_ref[...] = e / jnp.sum(e, axis=-1, keepdims=True)


def _row_spec():
    return pl.BlockSpec((RB, D), lambda b: (b, 0))


def _col_spec():
    return pl.BlockSpec((RB, 1), lambda b: (b, 0))


def _full_spec(shape):
    return pl.BlockSpec(shape, lambda b: tuple(0 for _ in shape))


def kernel(x, edge_index, batch, W_in, b_in, W_g0, b_g0, W_g1, b_g1,
           W_g2, b_g2, W_c1, b_c1, W_c2, b_c2):
    src = edge_index[0]
    dst = edge_index[1]

    hist = _sc_degree(dst)
    h0 = hist[:N_NODES].reshape(N_NODES, 1)
    h1 = hist[NPAD:NPAD + N_NODES].reshape(N_NODES, 1)

    tc_in = pl.pallas_call(
        _tc_in_body,
        grid=(NGRID,),
        in_specs=[_row_spec(), _full_spec((D, D)), _full_spec((1, D)),
                  _full_spec((D, D)), _col_spec(), _col_spec()],
        out_specs=[_row_spec(), _row_spec()],
        out_shape=[jax.ShapeDtypeStruct((N_NODES, D), jnp.float32)] * 2,
    )
    h, g = tc_in(x, W_in, b_in.reshape(1, D), W_g0, h0, h1)

    tc_mid = pl.pallas_call(
        _tc_mid_body,
        grid=(NGRID,),
        in_specs=[_row_spec(), _row_spec(), _row_spec(), _row_spec(),
                  _col_spec(), _col_spec(), _full_spec((1, D)),
                  _full_spec((D, D))],
        out_specs=[_row_spec(), _row_spec()],
        out_shape=[jax.ShapeDtypeStruct((N_NODES, D), jnp.float32)] * 2,
    )

    for (b_l, W_next) in ((b_g0, W_g1), (b_g1, W_g2)):
        part = _sc_scatter(g, src, dst)
        p0 = part[0, :N_NODES]
        p1 = part[1, :N_NODES]
        h, g = tc_mid(h, g, p0, p1, h0, h1, b_l.reshape(1, D), W_next)

    part = _sc_scatter(g, src, dst)
    p0 = part[0, :N_NODES]
    p1 = part[1, :N_NODES]

    tc_fin = pl.pallas_call(
        _tc_fin_body,
        grid=(NGRID,),
        in_specs=[_row_spec(), _row_spec(), _row_spec(), _row_spec(),
                  _col_spec(), _col_spec(), _full_spec((1, D)),
                  _col_spec(), _full_spec((D, D)), _full_spec((1, D)),
                  _full_spec((D, N_CLASSES)), _full_spec((1, N_CLASSES))],
        out_specs=pl.BlockSpec((N_GRAPHS, N_CLASSES), lambda b: (0, 0)),
        out_shape=jax.ShapeDtypeStruct((N_GRAPHS, N_CLASSES), jnp.float32),
        scratch_shapes=[pltpu.VMEM((N_GRAPHS, D), jnp.float32)],
    )
    logits = tc_fin(h, g, p0, p1, h0, h1, b_g2.reshape(1, D),
                    batch.reshape(N_NODES, 1), W_c1, b_c1.reshape(1, D),
                    W_c2, b_c2.reshape(1, N_CLASSES))
    return logits
